# baseline (speedup 1.0000x reference)
import functools
import jax
import jax.numpy as jnp
from jax import lax
from jax.experimental import pallas as pl
from jax.experimental.pallas import tpu as pltpu
from jax.experimental.pallas import tpu_sc as plsc

N = 10000
E = 320000
D_IN = 128
D_HID = 128
D_OUT = 40
D_OUT_PAD = 128

NC = 2
NS = 16
NW = NC * NS

B = 64
CH = 160
B_DEG = 128
CH_DEG = 80
EPW = B * CH
E_PAD = NW * EPW
N_PAD = 10240
RPS = N_PAD // NS

_mesh = plsc.VectorSubcoreMesh(core_axis_name="c", subcore_axis_name="s")


DEG_W = 128


def _fill_vmem_rows(buf, nrows, width, vec16):
    def frow(r, carry):
        for k in range(width // 16):
            buf[r, pl.ds(k * 16, 16)] = vec16
        return carry

    lax.fori_loop(0, nrows, frow, 0)


def _deg_body(src_hbm, dst_hbm, dP_hbm, sidx_v, didx_v, ones_v, acc, sem):
    c = lax.axis_index("c")
    s = lax.axis_index("s")
    w = s * NC + c

    pltpu.sync_copy(src_hbm.at[w], sidx_v)
    pltpu.sync_copy(dst_hbm.at[w], didx_v)

    def zacc(j, carry):
        pltpu.sync_copy(ones_v, acc.at[pl.ds(s * RPS + j * B_DEG, B_DEG)])
        return carry

    for phase in range(2):
        idx_v = sidx_v if phase == 0 else didx_v
        _fill_vmem_rows(ones_v, B_DEG, DEG_W, jnp.zeros((16,), jnp.float32))
        lax.fori_loop(0, RPS // B_DEG, zacc, 0)
        _fill_vmem_rows(ones_v, B_DEG, DEG_W, jnp.ones((16,), jnp.float32))
        plsc.subcore_barrier()

        def body(j, carry):
            pltpu.sync_copy(ones_v, acc.at[idx_v.at[j]], add=True)
            return carry

        lax.fori_loop(0, CH_DEG, body, 0)
        plsc.subcore_barrier()
        pltpu.sync_copy(
            acc.at[pl.ds(s * RPS, RPS)], dP_hbm.at[c, phase, pl.ds(s * RPS, RPS)]
        )
        plsc.subcore_barrier()


_deg_call = pl.kernel(
    _deg_body,
    out_type=jax.ShapeDtypeStruct((NC, 2, N_PAD, DEG_W), jnp.float32),
    mesh=_mesh,
    scratch_types=[
        pltpu.VMEM((CH_DEG, B_DEG), jnp.int32),
        pltpu.VMEM((CH_DEG, B_DEG), jnp.int32),
        pltpu.VMEM((B_DEG, DEG_W), jnp.float32),
        pltpu.VMEM_SHARED((N_PAD, DEG_W), jnp.float32),
        pltpu.SemaphoreType.DMA,
    ],
)


def _agg_body(D, h_hbm, src_hbm, dst_hbm, z_hbm, out_hbm, sidx_v, didx_v, rows0, rows1, acc, sem0, sem1):
    c = lax.axis_index("c")
    s = lax.axis_index("s")
    w = s * NC + c
    rows = [rows0, rows1]
    sems = [sem0, sem1]

    pltpu.sync_copy(z_hbm, acc.at[pl.ds(s * RPS, RPS)])

    pltpu.sync_copy(src_hbm.at[w], sidx_v)
    pltpu.sync_copy(dst_hbm.at[w], didx_v)
    plsc.subcore_barrier()

    def _gidx(j):
        return sidx_v.at[pl.ds(pl.multiple_of(j * B, B), B)]

    for b in range(2):
        pltpu.async_copy(h_hbm.at[_gidx(b)], rows[b], sems[b])

    def body(i, carry):
        for b in range(2):
            j = i * 2 + b
            pltpu.make_async_copy(h_hbm.at[_gidx(j)], rows[b], sems[b]).wait()
            pltpu.sync_copy(rows[b], acc.at[didx_v.at[j]], add=True)
            pltpu.async_copy(h_hbm.at[_gidx(j + 2)], rows[b], sems[b])
        return carry

    lax.fori_loop(0, CH // 2 - 1, body, 0)
    for b in range(2):
        j = CH - 2 + b
        pltpu.make_async_copy(h_hbm.at[_gidx(j)], rows[b], sems[b]).wait()
        pltpu.sync_copy(rows[b], acc.at[didx_v.at[j]], add=True)

    plsc.subcore_barrier()

    pltpu.sync_copy(acc.at[pl.ds(s * RPS, RPS)], out_hbm.at[c, pl.ds(s * RPS, RPS)])


def _make_agg(D):
    return pl.kernel(
        functools.partial(_agg_body, D),
        out_type=jax.ShapeDtypeStruct((NC, N_PAD, D), jnp.float32),
        mesh=_mesh,
        scratch_types=[
            pltpu.VMEM((EPW,), jnp.int32),
            pltpu.VMEM((CH, B), jnp.int32),
            pltpu.VMEM((B, D), jnp.float32),
            pltpu.VMEM((B, D), jnp.float32),
            pltpu.VMEM_SHARED((N_PAD, D), jnp.float32),
            pltpu.SemaphoreType.DMA,
            pltpu.SemaphoreType.DMA,
        ],
    )


_agg128 = _make_agg(128)


def _norm_body(dP_ref, s1_ref, s23_ref, sf_ref):
    do = dP_ref[0, 0, :, 0:1] + dP_ref[1, 0, :, 0:1]
    di = dP_ref[0, 1, :, 0:1] + dP_ref[1, 1, :, 0:1]
    ns = jnp.where(do > 0, lax.rsqrt(jnp.maximum(do, 1.0)), 0.0)
    nd = jnp.where(di > 0, lax.rsqrt(jnp.maximum(di, 1.0)), 0.0)
    s1_ref[...] = ns
    s23_ref[...] = ns * nd
    sf_ref[...] = nd


RB = 2048

_norm_call = pl.pallas_call(
    _norm_body,
    grid=(N_PAD // RB,),
    in_specs=[pl.BlockSpec((NC, 2, RB, DEG_W), lambda i: (0, 0, i, 0))],
    out_specs=[
        pl.BlockSpec((RB, 1), lambda i: (i, 0)),
        pl.BlockSpec((RB, 1), lambda i: (i, 0)),
        pl.BlockSpec((RB, 1), lambda i: (i, 0)),
    ],
    out_shape=[
        jax.ShapeDtypeStruct((N_PAD, 1), jnp.float32),
        jax.ShapeDtypeStruct((N_PAD, 1), jnp.float32),
        jax.ShapeDtypeStruct((N_PAD, 1), jnp.float32),
    ],
)


def _mm1_body(x_ref, s_ref, w_ref, o_ref):
    o_ref[...] = jnp.dot(
        x_ref[...] * s_ref[...], w_ref[...], preferred_element_type=jnp.float32
    )


def _mm2_body(p_ref, s_ref, w_ref, o_ref):
    x = (p_ref[0] + p_ref[1]) * s_ref[...]
    o_ref[...] = jnp.dot(x, w_ref[...], preferred_element_type=jnp.float32)


def _final_body(p_ref, s_ref, o_ref):
    o_ref[...] = (p_ref[0] + p_ref[1]) * s_ref[...]


def _mm1_call(x, s, w):
    grid = (N_PAD // RB,)
    return pl.pallas_call(
        _mm1_body,
        grid=grid,
        in_specs=[
            pl.BlockSpec((RB, D_IN), lambda i: (i, 0)),
            pl.BlockSpec((RB, 1), lambda i: (i, 0)),
            pl.BlockSpec((D_IN, D_HID), lambda i: (0, 0)),
        ],
        out_specs=pl.BlockSpec((RB, D_HID), lambda i: (i, 0)),
        out_shape=jax.ShapeDtypeStruct((N_PAD, D_HID), jnp.float32),
    )(x, s, w)


def _mm2_call(p, s, w, dout):
    grid = (N_PAD // RB,)
    din = p.shape[-1]
    return pl.pallas_call(
        _mm2_body,
        grid=grid,
        in_specs=[
            pl.BlockSpec((2, RB, din), lambda i: (0, i, 0)),
            pl.BlockSpec((RB, 1), lambda i: (i, 0)),
            pl.BlockSpec((din, dout), lambda i: (0, 0)),
        ],
        out_specs=pl.BlockSpec((RB, dout), lambda i: (i, 0)),
        out_shape=jax.ShapeDtypeStruct((N_PAD, dout), jnp.float32),
    )(p, s, w)


def _final_call(p, s):
    grid = (N_PAD // RB,)
    return pl.pallas_call(
        _final_body,
        grid=grid,
        in_specs=[
            pl.BlockSpec((2, RB, D_OUT_PAD), lambda i: (0, i, 0)),
            pl.BlockSpec((RB, 1), lambda i: (i, 0)),
        ],
        out_specs=pl.BlockSpec((RB, D_OUT_PAD), lambda i: (i, 0)),
        out_shape=jax.ShapeDtypeStruct((N_PAD, D_OUT_PAD), jnp.float32),
    )(p, s)


def kernel(features, edge_index, W1, W2, W3):
    pad_idx = jnp.full((E_PAD - E,), N_PAD - 1, jnp.int32)
    src_flat = jnp.concatenate([edge_index[0], pad_idx])
    dst_flat = jnp.concatenate([edge_index[1], pad_idx])
    srcp = src_flat.reshape(NW, EPW)
    dstp = dst_flat.reshape(NW, CH, B)
    src_deg = src_flat.reshape(NW, CH_DEG, B_DEG)
    dst_deg = dst_flat.reshape(NW, CH_DEG, B_DEG)
    xp = jnp.pad(features, ((0, N_PAD - N), (0, 0)))
    w3p = jnp.pad(W3, ((0, 0), (0, D_OUT_PAD - D_OUT)))

    zrows = jnp.zeros((RPS, 128), jnp.float32)

    dP = _deg_call(src_deg, dst_deg)
    s1, s23, sf = _norm_call(dP)

    h = _mm1_call(xp, s1, W1)
    p = _agg128(h, srcp, dstp, zrows)
    h = _mm2_call(p, s23, W2, D_HID)
    p = _agg128(h, srcp, dstp, zrows)
    h = _mm2_call(p, s23, w3p, D_OUT_PAD)
    p = _agg128(h, srcp, dstp, zrows)
    out = _final_call(p, sf)
    return out[:N, :D_OUT]


# speedup vs baseline: 2.2092x; 2.2092x over previous
import functools
import jax
import jax.numpy as jnp
from jax import lax
from jax.experimental import pallas as pl
from jax.experimental.pallas import tpu as pltpu
from jax.experimental.pallas import tpu_sc as plsc

N = 10000
E = 320000
D_IN = 128
D_HID = 128
D_OUT = 40
D_OUT_PAD = 128

NC = 2
NS = 16
NW = NC * NS

B = 64
CH = 160
B_DEG = 128
CH_DEG = 80
EPW = B * CH
E_PAD = NW * EPW
N_PAD = 10240
RPS = N_PAD // NS

_mesh = plsc.VectorSubcoreMesh(core_axis_name="c", subcore_axis_name="s")


DEG_W = 128


def _fill_vmem_rows(buf, nrows, width, vec16):
    def frow(r, carry):
        for k in range(width // 16):
            buf[r, pl.ds(k * 16, 16)] = vec16
        return carry

    lax.fori_loop(0, nrows, frow, 0)


def _deg_body(src_hbm, dst_hbm, dP_hbm, sidx_v, didx_v, ones_v, acc, sem):
    c = lax.axis_index("c")
    s = lax.axis_index("s")
    w = s * NC + c

    pltpu.sync_copy(src_hbm.at[w], sidx_v)
    pltpu.sync_copy(dst_hbm.at[w], didx_v)

    def zacc(j, carry):
        pltpu.sync_copy(ones_v, acc.at[pl.ds(s * RPS + j * B_DEG, B_DEG)])
        return carry

    for phase in range(2):
        idx_v = sidx_v if phase == 0 else didx_v
        _fill_vmem_rows(ones_v, B_DEG, DEG_W, jnp.zeros((16,), jnp.float32))
        lax.fori_loop(0, RPS // B_DEG, zacc, 0)
        _fill_vmem_rows(ones_v, B_DEG, DEG_W, jnp.ones((16,), jnp.float32))
        plsc.subcore_barrier()

        def body(j, carry):
            pltpu.sync_copy(ones_v, acc.at[idx_v.at[j]], add=True)
            return carry

        lax.fori_loop(0, CH_DEG, body, 0)
        plsc.subcore_barrier()
        pltpu.sync_copy(
            acc.at[pl.ds(s * RPS, RPS)], dP_hbm.at[c, phase, pl.ds(s * RPS, RPS)]
        )
        plsc.subcore_barrier()


_deg_call = pl.kernel(
    _deg_body,
    out_type=jax.ShapeDtypeStruct((NC, 2, N_PAD, DEG_W), jnp.float32),
    mesh=_mesh,
    scratch_types=[
        pltpu.VMEM((CH_DEG, B_DEG), jnp.int32),
        pltpu.VMEM((CH_DEG, B_DEG), jnp.int32),
        pltpu.VMEM((B_DEG, DEG_W), jnp.float32),
        pltpu.VMEM_SHARED((N_PAD, DEG_W), jnp.float32),
        pltpu.SemaphoreType.DMA,
    ],
)


def _agg_body(D, h_hbm, src_hbm, dst_hbm, z_hbm, out_hbm, sidx_v, didx_v, rows0, rows1, acc, sem0, sem1):
    c = lax.axis_index("c")
    s = lax.axis_index("s")
    w = s * NC + c
    rows = [rows0, rows1]
    sems = [sem0, sem1]

    pltpu.sync_copy(z_hbm, acc.at[pl.ds(s * RPS, RPS)])

    pltpu.sync_copy(src_hbm.at[w], sidx_v)
    pltpu.sync_copy(dst_hbm.at[w], didx_v)
    plsc.subcore_barrier()

    def _gidx(j):
        return sidx_v.at[pl.ds(pl.multiple_of(j * B, B), B)]

    for b in range(2):
        pltpu.async_copy(h_hbm.at[_gidx(b)], rows[b], sems[b])

    def body(i, carry):
        for b in range(2):
            j = i * 2 + b
            pltpu.make_async_copy(h_hbm.at[_gidx(j)], rows[b], sems[b]).wait()
            pltpu.sync_copy(rows[b], acc.at[didx_v.at[j]], add=True)
            pltpu.async_copy(h_hbm.at[_gidx(j + 2)], rows[b], sems[b])
        return carry

    lax.fori_loop(0, CH // 2 - 1, body, 0)
    for b in range(2):
        j = CH - 2 + b
        pltpu.make_async_copy(h_hbm.at[_gidx(j)], rows[b], sems[b]).wait()
        pltpu.sync_copy(rows[b], acc.at[didx_v.at[j]], add=True)

    plsc.subcore_barrier()

    pltpu.sync_copy(acc.at[pl.ds(s * RPS, RPS)], out_hbm.at[c, pl.ds(s * RPS, RPS)])


def _make_agg(D):
    return pl.kernel(
        functools.partial(_agg_body, D),
        out_type=jax.ShapeDtypeStruct((NC, N_PAD, D), jnp.float32),
        mesh=_mesh,
        scratch_types=[
            pltpu.VMEM((EPW,), jnp.int32),
            pltpu.VMEM((CH, B), jnp.int32),
            pltpu.VMEM((B, D), jnp.float32),
            pltpu.VMEM((B, D), jnp.float32),
            pltpu.VMEM_SHARED((N_PAD, D), jnp.float32),
            pltpu.SemaphoreType.DMA,
            pltpu.SemaphoreType.DMA,
        ],
    )


_agg128 = _make_agg(128)


def _norm_body(dP_ref, s1_ref, s23_ref, sf_ref):
    do = dP_ref[0, 0, :, 0:1] + dP_ref[1, 0, :, 0:1]
    di = dP_ref[0, 1, :, 0:1] + dP_ref[1, 1, :, 0:1]
    ns = jnp.where(do > 0, lax.rsqrt(jnp.maximum(do, 1.0)), 0.0)
    nd = jnp.where(di > 0, lax.rsqrt(jnp.maximum(di, 1.0)), 0.0)
    s1_ref[...] = ns
    s23_ref[...] = ns * nd
    sf_ref[...] = nd


RB = 2048

_norm_call = pl.pallas_call(
    _norm_body,
    grid=(N_PAD // RB,),
    in_specs=[pl.BlockSpec((NC, 2, RB, DEG_W), lambda i: (0, 0, i, 0))],
    out_specs=[
        pl.BlockSpec((RB, 1), lambda i: (i, 0)),
        pl.BlockSpec((RB, 1), lambda i: (i, 0)),
        pl.BlockSpec((RB, 1), lambda i: (i, 0)),
    ],
    out_shape=[
        jax.ShapeDtypeStruct((N_PAD, 1), jnp.float32),
        jax.ShapeDtypeStruct((N_PAD, 1), jnp.float32),
        jax.ShapeDtypeStruct((N_PAD, 1), jnp.float32),
    ],
)


def _mm1_body(x_ref, s_ref, w_ref, o_ref):
    o_ref[...] = jnp.dot(
        x_ref[...] * s_ref[...], w_ref[...], preferred_element_type=jnp.float32
    )


def _mm2_body(p_ref, s_ref, w_ref, o_ref):
    x = (p_ref[0] + p_ref[1]) * s_ref[...]
    o_ref[...] = jnp.dot(x, w_ref[...], preferred_element_type=jnp.float32)


def _final_body(p_ref, s_ref, o_ref):
    o_ref[...] = (p_ref[0] + p_ref[1]) * s_ref[...]


def _mm1_call(x, s, w):
    grid = (N_PAD // RB,)
    return pl.pallas_call(
        _mm1_body,
        grid=grid,
        in_specs=[
            pl.BlockSpec((RB, D_IN), lambda i: (i, 0)),
            pl.BlockSpec((RB, 1), lambda i: (i, 0)),
            pl.BlockSpec((D_IN, D_HID), lambda i: (0, 0)),
        ],
        out_specs=pl.BlockSpec((RB, D_HID), lambda i: (i, 0)),
        out_shape=jax.ShapeDtypeStruct((N_PAD, D_HID), jnp.float32),
    )(x, s, w)


def _mm2_call(p, s, w, dout):
    grid = (N_PAD // RB,)
    din = p.shape[-1]
    return pl.pallas_call(
        _mm2_body,
        grid=grid,
        in_specs=[
            pl.BlockSpec((2, RB, din), lambda i: (0, i, 0)),
            pl.BlockSpec((RB, 1), lambda i: (i, 0)),
            pl.BlockSpec((din, dout), lambda i: (0, 0)),
        ],
        out_specs=pl.BlockSpec((RB, dout), lambda i: (i, 0)),
        out_shape=jax.ShapeDtypeStruct((N_PAD, dout), jnp.float32),
    )(p, s, w)


def _final_call(p, s):
    grid = (N_PAD // RB,)
    return pl.pallas_call(
        _final_body,
        grid=grid,
        in_specs=[
            pl.BlockSpec((2, RB, D_OUT_PAD), lambda i: (0, i, 0)),
            pl.BlockSpec((RB, 1), lambda i: (i, 0)),
        ],
        out_specs=pl.BlockSpec((RB, D_OUT_PAD), lambda i: (i, 0)),
        out_shape=jax.ShapeDtypeStruct((N_PAD, D_OUT_PAD), jnp.float32),
    )(p, s)


def kernel(features, edge_index, W1, W2, W3):
    pad_idx = N + jnp.arange(E_PAD - E, dtype=jnp.int32) % (N_PAD - N)
    src_flat = jnp.concatenate([edge_index[0], pad_idx])
    dst_flat = jnp.concatenate([edge_index[1], pad_idx])
    srcp = src_flat.reshape(NW, EPW)
    dstp = dst_flat.reshape(NW, CH, B)
    src_deg = src_flat.reshape(NW, CH_DEG, B_DEG)
    dst_deg = dst_flat.reshape(NW, CH_DEG, B_DEG)
    xp = jnp.pad(features, ((0, N_PAD - N), (0, 0)))
    w3p = jnp.pad(W3, ((0, 0), (0, D_OUT_PAD - D_OUT)))

    zrows = jnp.zeros((RPS, 128), jnp.float32)

    dP = _deg_call(src_deg, dst_deg)
    s1, s23, sf = _norm_call(dP)

    h = _mm1_call(xp, s1, W1)
    p = _agg128(h, srcp, dstp, zrows)
    h = _mm2_call(p, s23, W2, D_HID)
    p = _agg128(h, srcp, dstp, zrows)
    h = _mm2_call(p, s23, w3p, D_OUT_PAD)
    p = _agg128(h, srcp, dstp, zrows)
    out = _final_call(p, sf)
    return out[:N, :D_OUT]


# speedup vs baseline: 2.4548x; 1.1111x over previous
import functools
import jax
import jax.numpy as jnp
from jax import lax
from jax.experimental import pallas as pl
from jax.experimental.pallas import tpu as pltpu
from jax.experimental.pallas import tpu_sc as plsc

N = 10000
E = 320000
D_IN = 128
D_HID = 128
D_OUT = 40
D_OUT_PAD = 128

NC = 2
NS = 16
NW = NC * NS

B = 128
CH = 80
CH_H = CH // 2
B_DEG = 128
CH_DEG = 80
EPW = B * CH
E_PAD = NW * EPW
N_PAD = 10240
RPS = N_PAD // NS

_mesh = plsc.VectorSubcoreMesh(core_axis_name="c", subcore_axis_name="s")


DEG_W = 128


def _fill_vmem_rows(buf, nrows, width, vec16):
    def frow(r, carry):
        for k in range(width // 16):
            buf[r, pl.ds(k * 16, 16)] = vec16
        return carry

    lax.fori_loop(0, nrows, frow, 0)


def _deg_body(src_hbm, dst_hbm, dP_hbm, sidx_v, didx_v, ones_v, acc, sem):
    c = lax.axis_index("c")
    s = lax.axis_index("s")
    w = s * NC + c

    pltpu.sync_copy(src_hbm.at[w], sidx_v)
    pltpu.sync_copy(dst_hbm.at[w], didx_v)

    def zacc(j, carry):
        pltpu.sync_copy(ones_v, acc.at[pl.ds(s * RPS + j * B_DEG, B_DEG)])
        return carry

    for phase in range(2):
        idx_v = sidx_v if phase == 0 else didx_v
        _fill_vmem_rows(ones_v, B_DEG, DEG_W, jnp.zeros((16,), jnp.float32))
        lax.fori_loop(0, RPS // B_DEG, zacc, 0)
        _fill_vmem_rows(ones_v, B_DEG, DEG_W, jnp.ones((16,), jnp.float32))
        plsc.subcore_barrier()

        def body(j, carry):
            pltpu.sync_copy(ones_v, acc.at[idx_v.at[j]], add=True)
            return carry

        lax.fori_loop(0, CH_DEG, body, 0)
        plsc.subcore_barrier()
        pltpu.sync_copy(
            acc.at[pl.ds(s * RPS, RPS)], dP_hbm.at[c, phase, pl.ds(s * RPS, RPS)]
        )
        plsc.subcore_barrier()


_deg_call = pl.kernel(
    _deg_body,
    out_type=jax.ShapeDtypeStruct((NC, 2, N_PAD, DEG_W), jnp.float32),
    mesh=_mesh,
    scratch_types=[
        pltpu.VMEM((CH_DEG, B_DEG), jnp.int32),
        pltpu.VMEM((CH_DEG, B_DEG), jnp.int32),
        pltpu.VMEM((B_DEG, DEG_W), jnp.float32),
        pltpu.VMEM_SHARED((N_PAD, DEG_W), jnp.float32),
        pltpu.SemaphoreType.DMA,
    ],
)


def _agg_body(D, h_hbm, src_hbm, dst_hbm, z_hbm, out_hbm, sidx_v, didx_v, rows0, rows1, acc, sem0, sem1):
    c = lax.axis_index("c")
    s = lax.axis_index("s")
    w = s * NC + c
    rows = [rows0, rows1]
    sems = [sem0, sem1]

    pltpu.sync_copy(z_hbm, acc.at[pl.ds(s * RPS, RPS)])

    pltpu.sync_copy(dst_hbm.at[w], didx_v)
    plsc.subcore_barrier()

    def _gidx(j):
        return sidx_v.at[pl.ds(pl.multiple_of(j * B, B), B)]

    for half in range(2):
        pltpu.sync_copy(
            src_hbm.at[w, pl.ds(half * CH_H * B, CH_H * B)], sidx_v
        )

        for b in range(2):
            pltpu.async_copy(h_hbm.at[_gidx(b)], rows[b], sems[b])

        def body(i, carry):
            for b in range(2):
                j = i * 2 + b
                pltpu.make_async_copy(h_hbm.at[_gidx(j)], rows[b], sems[b]).wait()
                pltpu.sync_copy(
                    rows[b], acc.at[didx_v.at[half * CH_H + j]], add=True
                )
                pltpu.async_copy(h_hbm.at[_gidx(j + 2)], rows[b], sems[b])
            return carry

        lax.fori_loop(0, CH_H // 2 - 1, body, 0)
        for b in range(2):
            j = CH_H - 2 + b
            pltpu.make_async_copy(h_hbm.at[_gidx(j)], rows[b], sems[b]).wait()
            pltpu.sync_copy(
                rows[b], acc.at[didx_v.at[half * CH_H + j]], add=True
            )

    plsc.subcore_barrier()

    pltpu.sync_copy(acc.at[pl.ds(s * RPS, RPS)], out_hbm.at[c, pl.ds(s * RPS, RPS)])


def _make_agg(D):
    return pl.kernel(
        functools.partial(_agg_body, D),
        out_type=jax.ShapeDtypeStruct((NC, N_PAD, D), jnp.float32),
        mesh=_mesh,
        scratch_types=[
            pltpu.VMEM((EPW // 2,), jnp.int32),
            pltpu.VMEM((CH, B), jnp.int32),
            pltpu.VMEM((B, D), jnp.float32),
            pltpu.VMEM((B, D), jnp.float32),
            pltpu.VMEM_SHARED((N_PAD, D), jnp.float32),
            pltpu.SemaphoreType.DMA,
            pltpu.SemaphoreType.DMA,
        ],
    )


_agg128 = _make_agg(128)


def _norm_body(dP_ref, s1_ref, s23_ref, sf_ref):
    do = dP_ref[0, 0, :, 0:1] + dP_ref[1, 0, :, 0:1]
    di = dP_ref[0, 1, :, 0:1] + dP_ref[1, 1, :, 0:1]
    ns = jnp.where(do > 0, lax.rsqrt(jnp.maximum(do, 1.0)), 0.0)
    nd = jnp.where(di > 0, lax.rsqrt(jnp.maximum(di, 1.0)), 0.0)
    s1_ref[...] = ns
    s23_ref[...] = ns * nd
    sf_ref[...] = nd


RB = 2048

_norm_call = pl.pallas_call(
    _norm_body,
    grid=(N_PAD // RB,),
    in_specs=[pl.BlockSpec((NC, 2, RB, DEG_W), lambda i: (0, 0, i, 0))],
    out_specs=[
        pl.BlockSpec((RB, 1), lambda i: (i, 0)),
        pl.BlockSpec((RB, 1), lambda i: (i, 0)),
        pl.BlockSpec((RB, 1), lambda i: (i, 0)),
    ],
    out_shape=[
        jax.ShapeDtypeStruct((N_PAD, 1), jnp.float32),
        jax.ShapeDtypeStruct((N_PAD, 1), jnp.float32),
        jax.ShapeDtypeStruct((N_PAD, 1), jnp.float32),
    ],
)


def _mm1_body(x_ref, s_ref, w_ref, o_ref):
    o_ref[...] = jnp.dot(
        x_ref[...] * s_ref[...], w_ref[...], preferred_element_type=jnp.float32
    )


def _mm2_body(p_ref, s_ref, w_ref, o_ref):
    x = (p_ref[0] + p_ref[1]) * s_ref[...]
    o_ref[...] = jnp.dot(x, w_ref[...], preferred_element_type=jnp.float32)


def _final_body(p_ref, s_ref, o_ref):
    o_ref[...] = (p_ref[0] + p_ref[1]) * s_ref[...]


def _mm1_call(x, s, w):
    grid = (N_PAD // RB,)
    return pl.pallas_call(
        _mm1_body,
        grid=grid,
        in_specs=[
            pl.BlockSpec((RB, D_IN), lambda i: (i, 0)),
            pl.BlockSpec((RB, 1), lambda i: (i, 0)),
            pl.BlockSpec((D_IN, D_HID), lambda i: (0, 0)),
        ],
        out_specs=pl.BlockSpec((RB, D_HID), lambda i: (i, 0)),
        out_shape=jax.ShapeDtypeStruct((N_PAD, D_HID), jnp.float32),
    )(x, s, w)


def _mm2_call(p, s, w, dout):
    grid = (N_PAD // RB,)
    din = p.shape[-1]
    return pl.pallas_call(
        _mm2_body,
        grid=grid,
        in_specs=[
            pl.BlockSpec((2, RB, din), lambda i: (0, i, 0)),
            pl.BlockSpec((RB, 1), lambda i: (i, 0)),
            pl.BlockSpec((din, dout), lambda i: (0, 0)),
        ],
        out_specs=pl.BlockSpec((RB, dout), lambda i: (i, 0)),
        out_shape=jax.ShapeDtypeStruct((N_PAD, dout), jnp.float32),
    )(p, s, w)


def _final_call(p, s):
    grid = (N_PAD // RB,)
    return pl.pallas_call(
        _final_body,
        grid=grid,
        in_specs=[
            pl.BlockSpec((2, RB, D_OUT_PAD), lambda i: (0, i, 0)),
            pl.BlockSpec((RB, 1), lambda i: (i, 0)),
        ],
        out_specs=pl.BlockSpec((RB, D_OUT_PAD), lambda i: (i, 0)),
        out_shape=jax.ShapeDtypeStruct((N_PAD, D_OUT_PAD), jnp.float32),
    )(p, s)


def kernel(features, edge_index, W1, W2, W3):
    pad_idx = N + jnp.arange(E_PAD - E, dtype=jnp.int32) % (N_PAD - N)
    src_flat = jnp.concatenate([edge_index[0], pad_idx])
    dst_flat = jnp.concatenate([edge_index[1], pad_idx])
    srcp = src_flat.reshape(NW, EPW)
    dstp = dst_flat.reshape(NW, CH, B)
    src_deg = src_flat.reshape(NW, CH_DEG, B_DEG)
    dst_deg = dst_flat.reshape(NW, CH_DEG, B_DEG)
    xp = jnp.pad(features, ((0, N_PAD - N), (0, 0)))
    w3p = jnp.pad(W3, ((0, 0), (0, D_OUT_PAD - D_OUT)))

    zrows = jnp.zeros((RPS, 128), jnp.float32)

    dP = _deg_call(src_deg, dst_deg)
    s1, s23, sf = _norm_call(dP)

    h = _mm1_call(xp, s1, W1)
    p = _agg128(h, srcp, dstp, zrows)
    h = _mm2_call(p, s23, W2, D_HID)
    p = _agg128(h, srcp, dstp, zrows)
    h = _mm2_call(p, s23, w3p, D_OUT_PAD)
    p = _agg128(h, srcp, dstp, zrows)
    out = _final_call(p, sf)
    return out[:N, :D_OUT]
